# Initial kernel scaffold; baseline (speedup 1.0000x reference)
import jax
import jax.numpy as jnp
from jax.experimental import pallas as pl


def kernel(x, adj, W0, b0, W1, b1):
    raise NotImplementedError("write your pallas kernel here")


# speedup vs baseline: 1.0095x; 1.0095x over previous
import functools

import jax
import jax.numpy as jnp
from jax.experimental import pallas as pl
from jax.experimental.pallas import tpu as pltpu

_N = 10000
_BM = 200


def _proj0_body(x_ref, w0_ref, o_ref):
    o_ref[...] = jnp.dot(
        x_ref[...].astype(jnp.bfloat16),
        w0_ref[...].astype(jnp.bfloat16),
        preferred_element_type=jnp.float32,
    ).astype(jnp.bfloat16)


def _layer0_body(adj_ref, v_ref, b0_ref, w1_ref, o_ref):
    acc = jnp.dot(
        adj_ref[...].astype(jnp.bfloat16),
        v_ref[...],
        preferred_element_type=jnp.float32,
    )
    h = jnp.maximum(acc + b0_ref[...], 0.0)
    o_ref[...] = jnp.dot(
        h.astype(jnp.bfloat16),
        w1_ref[...].astype(jnp.bfloat16),
        preferred_element_type=jnp.float32,
    ).astype(jnp.bfloat16)


def _layer1_body(adj_ref, v_ref, b1_ref, o_ref):
    o_ref[...] = (
        jnp.dot(
            adj_ref[...].astype(jnp.bfloat16),
            v_ref[...],
            preferred_element_type=jnp.float32,
        )
        + b1_ref[...]
    )


@functools.partial(jax.jit, donate_argnums=())
def kernel(x, adj, W0, b0, W1, b1):
    n, d_in = x.shape
    d_hid = W0.shape[1]
    d_out = W1.shape[1]
    b0r = b0.reshape(1, d_hid)
    b1r = b1.reshape(1, d_out)

    xw0 = pl.pallas_call(
        _proj0_body,
        out_shape=jax.ShapeDtypeStruct((n, d_hid), jnp.bfloat16),
    )(x, W0)

    gm = n // _BM
    hw1 = pl.pallas_call(
        _layer0_body,
        grid=(gm,),
        in_specs=[
            pl.BlockSpec((_BM, n), lambda i: (i, 0)),
            pl.BlockSpec((n, d_hid), lambda i: (0, 0)),
            pl.BlockSpec((1, d_hid), lambda i: (0, 0)),
            pl.BlockSpec((d_hid, d_out), lambda i: (0, 0)),
        ],
        out_specs=pl.BlockSpec((_BM, d_out), lambda i: (i, 0)),
        out_shape=jax.ShapeDtypeStruct((n, d_out), jnp.bfloat16),
        compiler_params=pltpu.CompilerParams(
            dimension_semantics=("arbitrary",),
        ),
    )(adj, xw0, b0r, W1)

    out = pl.pallas_call(
        _layer1_body,
        grid=(gm,),
        in_specs=[
            pl.BlockSpec((_BM, n), lambda i: (i, 0)),
            pl.BlockSpec((n, d_out), lambda i: (0, 0)),
            pl.BlockSpec((1, d_out), lambda i: (0, 0)),
        ],
        out_specs=pl.BlockSpec((_BM, d_out), lambda i: (i, 0)),
        out_shape=jax.ShapeDtypeStruct((n, d_out), jnp.float32),
        compiler_params=pltpu.CompilerParams(
            dimension_semantics=("arbitrary",),
        ),
    )(adj, hw1, b1r)

    return out
